# Initial kernel scaffold; baseline (speedup 1.0000x reference)
import jax
import jax.numpy as jnp
from jax.experimental import pallas as pl


def kernel(x, edge_index, batch, W1, b1, W2, b2, Wg, bg, Wm, bm, Wv, bv):
    raise NotImplementedError("write your pallas kernel here")


# speedup vs baseline: 12.9270x; 12.9270x over previous
import functools

import jax
import jax.numpy as jnp
from jax import lax
from jax.experimental import pallas as pl
from jax.experimental.pallas import tpu as pltpu
from jax.experimental.pallas import tpu_sc as plsc

N = 10000
E = 320000
D = 128
H = 128
L = 64
B = 16

NP = 10240
NTILES = 32
EPW = E // NTILES
CH = 80
NCH = EPW // CH
RPT = NP // 16

_MESH = plsc.VectorSubcoreMesh(core_axis_name="c", subcore_axis_name="s")


def _sc_degree(dst, ones_c, zeros_r):
    @functools.partial(
        pl.kernel,
        mesh=_MESH,
        out_type=jax.ShapeDtypeStruct((2, NP), jnp.float32),
        scratch_types=[
            pltpu.VMEM((CH,), jnp.int32),
            pltpu.VMEM((CH,), jnp.float32),
            pltpu.VMEM_SHARED((NP,), jnp.float32),
        ],
    )
    def k(dst_hbm, ones_hbm, zeros_hbm, out_hbm, idx_v, ones_v, acc_sh):
        c = lax.axis_index("c")
        s = lax.axis_index("s")
        w = c * 16 + s
        pltpu.sync_copy(ones_hbm, ones_v)
        pltpu.sync_copy(zeros_hbm, acc_sh.at[pl.ds(s * RPT, RPT)])
        plsc.subcore_barrier()
        base = w * EPW

        def body(i, carry):
            pltpu.sync_copy(dst_hbm.at[pl.ds(base + i * CH, CH)], idx_v)
            pltpu.sync_copy(ones_v, acc_sh.at[idx_v], add=True)
            return carry

        lax.fori_loop(0, NCH, body, 0)
        plsc.subcore_barrier()
        pltpu.sync_copy(acc_sh.at[pl.ds(s * RPT, RPT)],
                        out_hbm.at[c, pl.ds(s * RPT, RPT)])

    return k(dst, ones_c, zeros_r)


def _sc_edge_accum(src, dst, y, zeros_rows):
    @functools.partial(
        pl.kernel,
        mesh=_MESH,
        out_type=jax.ShapeDtypeStruct((2, NP, H), jnp.float32),
        scratch_types=[
            pltpu.VMEM((CH,), jnp.int32),
            pltpu.VMEM((CH,), jnp.int32),
            pltpu.VMEM((CH, H), jnp.float32),
            pltpu.VMEM_SHARED((NP, H), jnp.float32),
            pltpu.SemaphoreType.DMA,
        ],
    )
    def k(src_hbm, dst_hbm, y_hbm, zeros_hbm, out_hbm,
          idx_s, idx_d, rows_v, acc_sh, sem):
        c = lax.axis_index("c")
        s = lax.axis_index("s")
        w = c * 16 + s
        pltpu.sync_copy(zeros_hbm, acc_sh.at[pl.ds(s * RPT, RPT)])
        plsc.subcore_barrier()
        base = w * EPW

        def body(i, carry):
            off = base + i * CH
            pltpu.sync_copy(src_hbm.at[pl.ds(off, CH)], idx_s)
            pltpu.async_copy(y_hbm.at[idx_s], rows_v, sem).wait()
            pltpu.sync_copy(dst_hbm.at[pl.ds(off, CH)], idx_d)
            pltpu.sync_copy(rows_v, acc_sh.at[idx_d], add=True)
            return carry

        lax.fori_loop(0, NCH, body, 0)
        plsc.subcore_barrier()
        pltpu.sync_copy(acc_sh.at[pl.ds(s * RPT, RPT)],
                        out_hbm.at[c, pl.ds(s * RPT, RPT)])

    return k(src, dst, y, zeros_rows)


_RB = 1000
_NG = N // _RB


def _dinv_block(degp_b):
    return lax.rsqrt(degp_b[0] + degp_b[1] + 1.0)


def _tc_scale_matmul(x, W, degp3):
    def body(x_b, w_b, degp_b, y_b):
        dinv = _dinv_block(degp_b)
        y_b[...] = jnp.dot(x_b[...], w_b[...],
                           preferred_element_type=jnp.float32) * dinv

    return pl.pallas_call(
        body,
        grid=(_NG,),
        in_specs=[
            pl.BlockSpec((_RB, D), lambda i: (i, 0)),
            pl.BlockSpec((D, H), lambda i: (0, 0)),
            pl.BlockSpec((2, _RB, 1), lambda i: (0, i, 0)),
        ],
        out_specs=pl.BlockSpec((_RB, H), lambda i: (i, 0)),
        out_shape=jax.ShapeDtypeStruct((N, H), jnp.float32),
    )(x, W, degp3)


def _tc_conv1_out(acc, y1, degp3, b1, W2):
    def body(acc_b, y1_b, degp_b, b1_b, w2_b, y2_b):
        dinv = _dinv_block(degp_b)
        h = jnp.maximum(dinv * (acc_b[0] + acc_b[1] + y1_b[...]) + b1_b[...],
                        0.0)
        y2_b[...] = jnp.dot(h, w2_b[...],
                            preferred_element_type=jnp.float32) * dinv

    return pl.pallas_call(
        body,
        grid=(_NG,),
        in_specs=[
            pl.BlockSpec((2, _RB, H), lambda i: (0, i, 0)),
            pl.BlockSpec((_RB, H), lambda i: (i, 0)),
            pl.BlockSpec((2, _RB, 1), lambda i: (0, i, 0)),
            pl.BlockSpec((1, H), lambda i: (0, 0)),
            pl.BlockSpec((H, H), lambda i: (0, 0)),
        ],
        out_specs=pl.BlockSpec((_RB, H), lambda i: (i, 0)),
        out_shape=jax.ShapeDtypeStruct((N, H), jnp.float32),
    )(acc, y1, degp3, b1, W2)


def _tc_conv2_out(acc, y2, degp3, b2, Wg, bg):
    def body(acc_b, y2_b, degp_b, b2_b, wg_b, bg_b, h2_b, g_b):
        dinv = _dinv_block(degp_b)
        h2 = dinv * (acc_b[0] + acc_b[1] + y2_b[...]) + b2_b[...]
        h2_b[...] = h2
        g_b[...] = jnp.dot(h2, wg_b[...],
                           preferred_element_type=jnp.float32) + bg_b[...]

    return pl.pallas_call(
        body,
        grid=(_NG,),
        in_specs=[
            pl.BlockSpec((2, _RB, H), lambda i: (0, i, 0)),
            pl.BlockSpec((_RB, H), lambda i: (i, 0)),
            pl.BlockSpec((2, _RB, 1), lambda i: (0, i, 0)),
            pl.BlockSpec((1, H), lambda i: (0, 0)),
            pl.BlockSpec((H, 1), lambda i: (0, 0)),
            pl.BlockSpec((1, 1), lambda i: (0, 0)),
        ],
        out_specs=[
            pl.BlockSpec((_RB, H), lambda i: (i, 0)),
            pl.BlockSpec((_RB, 1), lambda i: (i, 0)),
        ],
        out_shape=[
            jax.ShapeDtypeStruct((N, H), jnp.float32),
            jax.ShapeDtypeStruct((N, 1), jnp.float32),
        ],
    )(acc, y2, degp3, b2, Wg, bg)


def _tc_pool(h2, g, batch2d, Wm, bm):
    def body(h2_r, g_r, batch_r, wm_r, bm_r, mu_r):
        mask = (batch_r[...] ==
                lax.broadcasted_iota(jnp.int32, (N, B), 1))
        maskf = mask.astype(jnp.float32)
        g = g_r[...]
        gm = jnp.max(jnp.where(mask, g, -1e30), axis=0, keepdims=True)
        gmax_node = jnp.sum(maskf * gm, axis=1, keepdims=True)
        ge = jnp.exp(g - gmax_node)
        gs = jnp.sum(maskf * ge, axis=0, keepdims=True)
        gs_node = jnp.sum(maskf * gs, axis=1, keepdims=True)
        gate = ge / gs_node
        wgt = maskf * gate
        pooled = lax.dot_general(wgt, h2_r[...],
                                 (((0,), (0,)), ((), ())),
                                 preferred_element_type=jnp.float32)
        mu_r[...] = jnp.dot(pooled, wm_r[...],
                            preferred_element_type=jnp.float32) + bm_r[...]

    return pl.pallas_call(
        body,
        in_specs=[
            pl.BlockSpec((N, H), lambda: (0, 0)),
            pl.BlockSpec((N, 1), lambda: (0, 0)),
            pl.BlockSpec((N, 1), lambda: (0, 0)),
            pl.BlockSpec((H, L), lambda: (0, 0)),
            pl.BlockSpec((1, L), lambda: (0, 0)),
        ],
        out_specs=pl.BlockSpec((B, L), lambda: (0, 0)),
        out_shape=jax.ShapeDtypeStruct((B, L), jnp.float32),
    )(h2, g, batch2d, Wm, bm)


def kernel(x, edge_index, batch, W1, b1, W2, b2, Wg, bg, Wm, bm, Wv, bv):
    del Wv, bv
    src = edge_index[0]
    dst = edge_index[1]

    ones_c = jnp.ones((CH,), jnp.float32)
    zeros_r = jnp.zeros((RPT,), jnp.float32)
    zeros_rows = jnp.zeros((RPT, H), jnp.float32)

    degp = _sc_degree(dst, ones_c, zeros_r)
    degp3 = degp.reshape(2, NP, 1)

    y1 = _tc_scale_matmul(x, W1, degp3)
    acc1 = _sc_edge_accum(src, dst, y1, zeros_rows)
    y2 = _tc_conv1_out(acc1, y1, degp3,
                       b1.reshape(1, H), W2)
    acc2 = _sc_edge_accum(src, dst, y2, zeros_rows)
    h2, g = _tc_conv2_out(acc2, y2, degp3, b2.reshape(1, H),
                          Wg, bg.reshape(1, 1))
    mu = _tc_pool(h2, g, batch.reshape(N, 1), Wm, bm.reshape(1, L))
    return mu


# speedup vs baseline: 24.5229x; 1.8970x over previous
import functools

import jax
import jax.numpy as jnp
from jax import lax
from jax.experimental import pallas as pl
from jax.experimental.pallas import tpu as pltpu
from jax.experimental.pallas import tpu_sc as plsc

N = 10000
E = 320000
D = 128
H = 128
L = 64
B = 16

NP = 10240
NTILES = 32
EPW = E // NTILES
CH = 80
NCH = EPW // CH
RPT = NP // 16

_MESH = plsc.VectorSubcoreMesh(core_axis_name="c", subcore_axis_name="s")


def _sc_degree(dst3, ones_c, zeros_r):
    @functools.partial(
        pl.kernel,
        mesh=_MESH,
        out_type=jax.ShapeDtypeStruct((2, NP), jnp.float32),
        scratch_types=[
            pltpu.VMEM((NCH, CH), jnp.int32),
            pltpu.VMEM((CH,), jnp.float32),
            pltpu.VMEM_SHARED((NP,), jnp.float32),
        ],
    )
    def k(dst_hbm, ones_hbm, zeros_hbm, out_hbm, idx_v, ones_v, acc_sh):
        c = lax.axis_index("c")
        s = lax.axis_index("s")
        w = c * 16 + s
        pltpu.sync_copy(ones_hbm, ones_v)
        pltpu.sync_copy(dst_hbm.at[w], idx_v)
        pltpu.sync_copy(zeros_hbm, acc_sh.at[pl.ds(s * RPT, RPT)])
        plsc.subcore_barrier()

        def body(i, carry):
            pltpu.sync_copy(ones_v, acc_sh.at[idx_v.at[i]], add=True)
            return carry

        lax.fori_loop(0, NCH, body, 0)
        plsc.subcore_barrier()
        pltpu.sync_copy(acc_sh.at[pl.ds(s * RPT, RPT)],
                        out_hbm.at[c, pl.ds(s * RPT, RPT)])

    return k(dst3, ones_c, zeros_r)


def _sc_edge_accum(eidx4, y, zeros_rows):
    @functools.partial(
        pl.kernel,
        mesh=_MESH,
        out_type=jax.ShapeDtypeStruct((2, NP, H), jnp.float32),
        scratch_types=[
            pltpu.VMEM((2, CH), jnp.int32),
            pltpu.VMEM((2, CH), jnp.int32),
            pltpu.VMEM((CH, H), jnp.float32),
            pltpu.VMEM((CH, H), jnp.float32),
            pltpu.VMEM_SHARED((NP, H), jnp.float32),
            pltpu.SemaphoreType.DMA,
            pltpu.SemaphoreType.DMA,
        ],
    )
    def k(eidx_hbm, y_hbm, zeros_hbm, out_hbm,
          idx0, idx1, rows0, rows1, acc_sh, sem0, sem1):
        c = lax.axis_index("c")
        s = lax.axis_index("s")
        w = c * 16 + s
        pltpu.sync_copy(zeros_hbm, acc_sh.at[pl.ds(s * RPT, RPT)])
        plsc.subcore_barrier()

        pltpu.sync_copy(eidx_hbm.at[w, 0], idx0)
        pltpu.async_copy(y_hbm.at[idx0.at[0]], rows0, sem0)

        def body(g, carry):
            a = 2 * g + 1
            pltpu.sync_copy(eidx_hbm.at[w, a], idx1)
            pltpu.async_copy(y_hbm.at[idx1.at[0]], rows1, sem1)
            pltpu.make_async_copy(y_hbm.at[idx0.at[0]], rows0, sem0).wait()
            pltpu.sync_copy(rows0, acc_sh.at[idx0.at[1]], add=True)
            pltpu.sync_copy(eidx_hbm.at[w, a + 1], idx0)
            pltpu.async_copy(y_hbm.at[idx0.at[0]], rows0, sem0)
            pltpu.make_async_copy(y_hbm.at[idx1.at[0]], rows1, sem1).wait()
            pltpu.sync_copy(rows1, acc_sh.at[idx1.at[1]], add=True)
            return carry

        lax.fori_loop(0, (NCH - 1) // 2, body, 0)
        pltpu.make_async_copy(y_hbm.at[idx0.at[0]], rows0, sem0).wait()
        pltpu.sync_copy(rows0, acc_sh.at[idx0.at[1]], add=True)
        plsc.subcore_barrier()
        pltpu.sync_copy(acc_sh.at[pl.ds(s * RPT, RPT)],
                        out_hbm.at[c, pl.ds(s * RPT, RPT)])

    return k(eidx4, y, zeros_rows)


_RB = 1000
_NG = N // _RB


def _dinv_block(degp_b):
    return lax.rsqrt(degp_b[0] + degp_b[1] + 1.0)


def _tc_scale_matmul(x, W, degp3):
    def body(x_b, w_b, degp_b, y_b):
        dinv = _dinv_block(degp_b)
        y_b[...] = jnp.dot(x_b[...], w_b[...],
                           preferred_element_type=jnp.float32) * dinv

    return pl.pallas_call(
        body,
        grid=(_NG,),
        in_specs=[
            pl.BlockSpec((_RB, D), lambda i: (i, 0)),
            pl.BlockSpec((D, H), lambda i: (0, 0)),
            pl.BlockSpec((2, _RB, 1), lambda i: (0, i, 0)),
        ],
        out_specs=pl.BlockSpec((_RB, H), lambda i: (i, 0)),
        out_shape=jax.ShapeDtypeStruct((N, H), jnp.float32),
    )(x, W, degp3)


def _tc_conv1_out(acc, y1, degp3, b1, W2):
    def body(acc_b, y1_b, degp_b, b1_b, w2_b, y2_b):
        dinv = _dinv_block(degp_b)
        h = jnp.maximum(dinv * (acc_b[0] + acc_b[1] + y1_b[...]) + b1_b[...],
                        0.0)
        y2_b[...] = jnp.dot(h, w2_b[...],
                            preferred_element_type=jnp.float32) * dinv

    return pl.pallas_call(
        body,
        grid=(_NG,),
        in_specs=[
            pl.BlockSpec((2, _RB, H), lambda i: (0, i, 0)),
            pl.BlockSpec((_RB, H), lambda i: (i, 0)),
            pl.BlockSpec((2, _RB, 1), lambda i: (0, i, 0)),
            pl.BlockSpec((1, H), lambda i: (0, 0)),
            pl.BlockSpec((H, H), lambda i: (0, 0)),
        ],
        out_specs=pl.BlockSpec((_RB, H), lambda i: (i, 0)),
        out_shape=jax.ShapeDtypeStruct((N, H), jnp.float32),
    )(acc, y1, degp3, b1, W2)


def _tc_conv2_out(acc, y2, degp3, b2, Wg, bg):
    def body(acc_b, y2_b, degp_b, b2_b, wg_b, bg_b, h2_b, g_b):
        dinv = _dinv_block(degp_b)
        h2 = dinv * (acc_b[0] + acc_b[1] + y2_b[...]) + b2_b[...]
        h2_b[...] = h2
        g_b[...] = jnp.dot(h2, wg_b[...],
                           preferred_element_type=jnp.float32) + bg_b[...]

    return pl.pallas_call(
        body,
        grid=(_NG,),
        in_specs=[
            pl.BlockSpec((2, _RB, H), lambda i: (0, i, 0)),
            pl.BlockSpec((_RB, H), lambda i: (i, 0)),
            pl.BlockSpec((2, _RB, 1), lambda i: (0, i, 0)),
            pl.BlockSpec((1, H), lambda i: (0, 0)),
            pl.BlockSpec((H, 1), lambda i: (0, 0)),
            pl.BlockSpec((1, 1), lambda i: (0, 0)),
        ],
        out_specs=[
            pl.BlockSpec((_RB, H), lambda i: (i, 0)),
            pl.BlockSpec((_RB, 1), lambda i: (i, 0)),
        ],
        out_shape=[
            jax.ShapeDtypeStruct((N, H), jnp.float32),
            jax.ShapeDtypeStruct((N, 1), jnp.float32),
        ],
    )(acc, y2, degp3, b2, Wg, bg)


def _tc_pool(h2, g, batch2d, Wm, bm):
    def body(h2_r, g_r, batch_r, wm_r, bm_r, mu_r):
        mask = (batch_r[...] ==
                lax.broadcasted_iota(jnp.int32, (N, B), 1))
        maskf = mask.astype(jnp.float32)
        g = g_r[...]
        gm = jnp.max(jnp.where(mask, g, -1e30), axis=0, keepdims=True)
        gmax_node = jnp.sum(maskf * gm, axis=1, keepdims=True)
        ge = jnp.exp(g - gmax_node)
        gs = jnp.sum(maskf * ge, axis=0, keepdims=True)
        gs_node = jnp.sum(maskf * gs, axis=1, keepdims=True)
        gate = ge / gs_node
        wgt = maskf * gate
        pooled = lax.dot_general(wgt, h2_r[...],
                                 (((0,), (0,)), ((), ())),
                                 preferred_element_type=jnp.float32)
        mu_r[...] = jnp.dot(pooled, wm_r[...],
                            preferred_element_type=jnp.float32) + bm_r[...]

    return pl.pallas_call(
        body,
        in_specs=[
            pl.BlockSpec((N, H), lambda: (0, 0)),
            pl.BlockSpec((N, 1), lambda: (0, 0)),
            pl.BlockSpec((N, 1), lambda: (0, 0)),
            pl.BlockSpec((H, L), lambda: (0, 0)),
            pl.BlockSpec((1, L), lambda: (0, 0)),
        ],
        out_specs=pl.BlockSpec((B, L), lambda: (0, 0)),
        out_shape=jax.ShapeDtypeStruct((B, L), jnp.float32),
    )(h2, g, batch2d, Wm, bm)


def kernel(x, edge_index, batch, W1, b1, W2, b2, Wg, bg, Wm, bm, Wv, bv):
    del Wv, bv
    src3 = edge_index[0].reshape(NTILES, NCH, CH)
    dst3 = edge_index[1].reshape(NTILES, NCH, CH)
    eidx4 = jnp.stack([src3, dst3], axis=2)

    ones_c = jnp.ones((CH,), jnp.float32)
    zeros_r = jnp.zeros((RPT,), jnp.float32)
    zeros_rows = jnp.zeros((RPT, H), jnp.float32)

    degp = _sc_degree(dst3, ones_c, zeros_r)
    degp3 = degp.reshape(2, NP, 1)

    y1 = _tc_scale_matmul(x, W1, degp3)
    acc1 = _sc_edge_accum(eidx4, y1, zeros_rows)
    y2 = _tc_conv1_out(acc1, y1, degp3,
                       b1.reshape(1, H), W2)
    acc2 = _sc_edge_accum(eidx4, y2, zeros_rows)
    h2, g = _tc_conv2_out(acc2, y2, degp3, b2.reshape(1, H),
                          Wg, bg.reshape(1, 1))
    mu = _tc_pool(h2, g, batch.reshape(N, 1), Wm, bm.reshape(1, L))
    return mu


# speedup vs baseline: 28.2207x; 1.1508x over previous
import functools

import jax
import jax.numpy as jnp
from jax import lax
from jax.experimental import pallas as pl
from jax.experimental.pallas import tpu as pltpu
from jax.experimental.pallas import tpu_sc as plsc

N = 10000
E = 320000
D = 128
H = 128
L = 64
B = 16

NP = 10240
NTILES = 32
EPW = E // NTILES
CH = 80
NCH = EPW // CH
RPT = NP // 16

_MESH = plsc.VectorSubcoreMesh(core_axis_name="c", subcore_axis_name="s")


def _sc_degree(dst3, ones_c, zeros_r):
    @functools.partial(
        pl.kernel,
        mesh=_MESH,
        out_type=jax.ShapeDtypeStruct((2, NP), jnp.float32),
        scratch_types=[
            pltpu.VMEM((NCH, CH), jnp.int32),
            pltpu.VMEM((CH,), jnp.float32),
            pltpu.VMEM_SHARED((NP,), jnp.float32),
        ],
    )
    def k(dst_hbm, ones_hbm, zeros_hbm, out_hbm, idx_v, ones_v, acc_sh):
        c = lax.axis_index("c")
        s = lax.axis_index("s")
        w = c * 16 + s
        pltpu.sync_copy(ones_hbm, ones_v)
        pltpu.sync_copy(dst_hbm.at[w], idx_v)
        pltpu.sync_copy(zeros_hbm, acc_sh.at[pl.ds(s * RPT, RPT)])
        plsc.subcore_barrier()

        def body(i, carry):
            pltpu.sync_copy(ones_v, acc_sh.at[idx_v.at[i]], add=True)
            return carry

        lax.fori_loop(0, NCH, body, 0)
        plsc.subcore_barrier()
        pltpu.sync_copy(acc_sh.at[pl.ds(s * RPT, RPT)],
                        out_hbm.at[c, pl.ds(s * RPT, RPT)])

    return k(dst3, ones_c, zeros_r)


def _sc_edge_accum(eidx4, y, zeros_rows):
    NB = 4

    @functools.partial(
        pl.kernel,
        mesh=_MESH,
        out_type=jax.ShapeDtypeStruct((2, NP, H), jnp.float32),
        scratch_types=(
            [pltpu.VMEM((2, CH), jnp.int32) for _ in range(NB)]
            + [pltpu.VMEM((CH, H), jnp.float32) for _ in range(NB)]
            + [pltpu.VMEM_SHARED((NP, H), jnp.float32)]
            + [pltpu.SemaphoreType.DMA] * (2 * NB)
        ),
    )
    def k(eidx_hbm, y_hbm, zeros_hbm, out_hbm, *refs):
        idx = refs[0:NB]
        rows = refs[NB:2 * NB]
        acc_sh = refs[2 * NB]
        gsem = refs[2 * NB + 1:3 * NB + 1]
        scsem = refs[3 * NB + 1:4 * NB + 1]
        c = lax.axis_index("c")
        s = lax.axis_index("s")
        w = c * 16 + s
        pltpu.sync_copy(zeros_hbm, acc_sh.at[pl.ds(s * RPT, RPT)])
        plsc.subcore_barrier()

        def load_and_gather(j, cc):
            pltpu.sync_copy(eidx_hbm.at[w, cc], idx[j])
            pltpu.async_copy(y_hbm.at[idx[j].at[0]], rows[j], gsem[j])

        def wait_gather(j):
            pltpu.make_async_copy(
                y_hbm.at[idx[j].at[0]], rows[j], gsem[j]).wait()

        def scatter(j):
            pltpu.async_copy(rows[j], acc_sh.at[idx[j].at[1]], scsem[j],
                             add=True)

        def wait_scatter(j):
            pltpu.make_async_copy(
                rows[j], acc_sh.at[idx[j].at[1]], scsem[j]).wait()

        for j in range(2):
            load_and_gather(j, j)
        for cc in range(4):
            j = cc % NB
            wait_gather(j)
            scatter(j)
            jj = (j + 2) % NB
            if cc >= 2:
                wait_scatter(jj)
            load_and_gather(jj, cc + 2)

        def body(g, carry):
            base = 4 * g
            for j in range(NB):
                wait_gather(j)
                scatter(j)
                jj = (j + 2) % NB
                wait_scatter(jj)
                load_and_gather(jj, base + j + 2)
            return carry

        lax.fori_loop(1, (NCH - 5) // 4, body, 0)

        for t in range(5):
            cc = NCH - 5 + t
            j = cc % NB
            wait_gather(j)
            scatter(j)
            if t < 3:
                jj = (j + 2) % NB
                wait_scatter(jj)
                load_and_gather(jj, cc + 2)
        for j in (1, 2, 3, 0):
            wait_scatter(j)

        plsc.subcore_barrier()
        pltpu.sync_copy(acc_sh.at[pl.ds(s * RPT, RPT)],
                        out_hbm.at[c, pl.ds(s * RPT, RPT)])

    return k(eidx4, y, zeros_rows)


_RB = 1000
_NG = N // _RB


def _dinv_block(degp_b):
    return lax.rsqrt(degp_b[0] + degp_b[1] + 1.0)


def _tc_scale_matmul(x, W, degp3):
    def body(x_b, w_b, degp_b, y_b):
        dinv = _dinv_block(degp_b)
        y_b[...] = jnp.dot(x_b[...], w_b[...],
                           preferred_element_type=jnp.float32) * dinv

    return pl.pallas_call(
        body,
        grid=(_NG,),
        in_specs=[
            pl.BlockSpec((_RB, D), lambda i: (i, 0)),
            pl.BlockSpec((D, H), lambda i: (0, 0)),
            pl.BlockSpec((2, _RB, 1), lambda i: (0, i, 0)),
        ],
        out_specs=pl.BlockSpec((_RB, H), lambda i: (i, 0)),
        out_shape=jax.ShapeDtypeStruct((N, H), jnp.float32),
    )(x, W, degp3)


def _tc_conv1_out(acc, y1, degp3, b1, W2):
    def body(acc_b, y1_b, degp_b, b1_b, w2_b, y2_b):
        dinv = _dinv_block(degp_b)
        h = jnp.maximum(dinv * (acc_b[0] + acc_b[1] + y1_b[...]) + b1_b[...],
                        0.0)
        y2_b[...] = jnp.dot(h, w2_b[...],
                            preferred_element_type=jnp.float32) * dinv

    return pl.pallas_call(
        body,
        grid=(_NG,),
        in_specs=[
            pl.BlockSpec((2, _RB, H), lambda i: (0, i, 0)),
            pl.BlockSpec((_RB, H), lambda i: (i, 0)),
            pl.BlockSpec((2, _RB, 1), lambda i: (0, i, 0)),
            pl.BlockSpec((1, H), lambda i: (0, 0)),
            pl.BlockSpec((H, H), lambda i: (0, 0)),
        ],
        out_specs=pl.BlockSpec((_RB, H), lambda i: (i, 0)),
        out_shape=jax.ShapeDtypeStruct((N, H), jnp.float32),
    )(acc, y1, degp3, b1, W2)


def _tc_conv2_pool(acc, y2, degp3, b2, Wg, bg, batch2d, Wm, bm):
    def body(acc_r, y2_r, degp_r, b2_r, wg_r, bg_r, batch_r, wm_r, bm_r,
             mu_r):
        dinv = lax.rsqrt(degp_r[0, 0:N, :] + degp_r[1, 0:N, :] + 1.0)
        h2 = (dinv * (acc_r[0, 0:N, :] + acc_r[1, 0:N, :] + y2_r[...])
              + b2_r[...])
        g = jnp.dot(h2, wg_r[...],
                    preferred_element_type=jnp.float32) + bg_r[...]
        mask = (batch_r[...] ==
                lax.broadcasted_iota(jnp.int32, (N, B), 1))
        maskf = mask.astype(jnp.float32)
        gm = jnp.max(jnp.where(mask, g, -1e30), axis=0, keepdims=True)
        gmax_node = jnp.sum(maskf * gm, axis=1, keepdims=True)
        ge = jnp.exp(g - gmax_node)
        gs = jnp.sum(maskf * ge, axis=0, keepdims=True)
        gs_node = jnp.sum(maskf * gs, axis=1, keepdims=True)
        gate = ge / gs_node
        wgt = maskf * gate
        pooled = lax.dot_general(wgt, h2,
                                 (((0,), (0,)), ((), ())),
                                 preferred_element_type=jnp.float32)
        mu_r[...] = jnp.dot(pooled, wm_r[...],
                            preferred_element_type=jnp.float32) + bm_r[...]

    return pl.pallas_call(
        body,
        in_specs=[
            pl.BlockSpec((2, NP, H), lambda: (0, 0, 0)),
            pl.BlockSpec((N, H), lambda: (0, 0)),
            pl.BlockSpec((2, NP, 1), lambda: (0, 0, 0)),
            pl.BlockSpec((1, H), lambda: (0, 0)),
            pl.BlockSpec((H, 1), lambda: (0, 0)),
            pl.BlockSpec((1, 1), lambda: (0, 0)),
            pl.BlockSpec((N, 1), lambda: (0, 0)),
            pl.BlockSpec((H, L), lambda: (0, 0)),
            pl.BlockSpec((1, L), lambda: (0, 0)),
        ],
        out_specs=pl.BlockSpec((B, L), lambda: (0, 0)),
        out_shape=jax.ShapeDtypeStruct((B, L), jnp.float32),
    )(acc, y2, degp3, b2, Wg, bg, batch2d, Wm, bm)


def kernel(x, edge_index, batch, W1, b1, W2, b2, Wg, bg, Wm, bm, Wv, bv):
    del Wv, bv
    src3 = edge_index[0].reshape(NTILES, NCH, CH)
    dst3 = edge_index[1].reshape(NTILES, NCH, CH)
    eidx4 = jnp.stack([src3, dst3], axis=2)

    ones_c = jnp.ones((CH,), jnp.float32)
    zeros_r = jnp.zeros((RPT,), jnp.float32)
    zeros_rows = jnp.zeros((RPT, H), jnp.float32)

    degp = _sc_degree(dst3, ones_c, zeros_r)
    degp3 = degp.reshape(2, NP, 1)

    y1 = _tc_scale_matmul(x, W1, degp3)
    acc1 = _sc_edge_accum(eidx4, y1, zeros_rows)
    y2 = _tc_conv1_out(acc1, y1, degp3,
                       b1.reshape(1, H), W2)
    acc2 = _sc_edge_accum(eidx4, y2, zeros_rows)
    mu = _tc_conv2_pool(acc2, y2, degp3, b2.reshape(1, H),
                        Wg, bg.reshape(1, 1),
                        batch.reshape(N, 1), Wm, bm.reshape(1, L))
    return mu


# speedup vs baseline: 28.3354x; 1.0041x over previous
import functools

import jax
import jax.numpy as jnp
from jax import lax
from jax.experimental import pallas as pl
from jax.experimental.pallas import tpu as pltpu
from jax.experimental.pallas import tpu_sc as plsc

N = 10000
E = 320000
D = 128
H = 128
L = 64
B = 16

NP = 10240
NTILES = 32
EPW = E // NTILES
CH = 80
NCH = EPW // CH
RPT = NP // 16

_MESH = plsc.VectorSubcoreMesh(core_axis_name="c", subcore_axis_name="s")


def _sc_degree(dst3, ones_c, zeros_r):
    @functools.partial(
        pl.kernel,
        mesh=_MESH,
        out_type=jax.ShapeDtypeStruct((2, NP), jnp.float32),
        scratch_types=[
            pltpu.VMEM((NCH, CH), jnp.int32),
            pltpu.VMEM((CH,), jnp.float32),
            pltpu.VMEM_SHARED((NP,), jnp.float32),
        ],
    )
    def k(dst_hbm, ones_hbm, zeros_hbm, out_hbm, idx_v, ones_v, acc_sh):
        c = lax.axis_index("c")
        s = lax.axis_index("s")
        w = c * 16 + s
        pltpu.sync_copy(ones_hbm, ones_v)
        pltpu.sync_copy(dst_hbm.at[w], idx_v)
        pltpu.sync_copy(zeros_hbm, acc_sh.at[pl.ds(s * RPT, RPT)])
        plsc.subcore_barrier()

        def body(i, carry):
            pltpu.sync_copy(ones_v, acc_sh.at[idx_v.at[i]], add=True)
            return carry

        lax.fori_loop(0, NCH, body, 0)
        plsc.subcore_barrier()
        pltpu.sync_copy(acc_sh.at[pl.ds(s * RPT, RPT)],
                        out_hbm.at[c, pl.ds(s * RPT, RPT)])

    return k(dst3, ones_c, zeros_r)


def _sc_edge_accum(eidx4, y, zeros_rows):
    NB = 4

    @functools.partial(
        pl.kernel,
        mesh=_MESH,
        out_type=jax.ShapeDtypeStruct((2, NP, H), jnp.float32),
        scratch_types=(
            [pltpu.VMEM((2, CH), jnp.int32) for _ in range(NB)]
            + [pltpu.VMEM((CH, H), jnp.float32) for _ in range(NB)]
            + [pltpu.VMEM_SHARED((NP, H), jnp.float32)]
            + [pltpu.SemaphoreType.DMA] * (2 * NB)
        ),
    )
    def k(eidx_hbm, y_hbm, zeros_hbm, out_hbm, *refs):
        idx = refs[0:NB]
        rows = refs[NB:2 * NB]
        acc_sh = refs[2 * NB]
        gsem = refs[2 * NB + 1:3 * NB + 1]
        scsem = refs[3 * NB + 1:4 * NB + 1]
        c = lax.axis_index("c")
        s = lax.axis_index("s")
        w = c * 16 + s

        def load_and_gather(j, cc):
            pltpu.sync_copy(eidx_hbm.at[w, cc], idx[j])
            pltpu.async_copy(y_hbm.at[idx[j].at[0]], rows[j], gsem[j])

        def wait_gather(j):
            pltpu.make_async_copy(
                y_hbm.at[idx[j].at[0]], rows[j], gsem[j]).wait()

        def scatter(j):
            pltpu.async_copy(rows[j], acc_sh.at[idx[j].at[1]], scsem[j],
                             add=True)

        def wait_scatter(j):
            pltpu.make_async_copy(
                rows[j], acc_sh.at[idx[j].at[1]], scsem[j]).wait()

        for j in range(2):
            load_and_gather(j, j)
        pltpu.sync_copy(zeros_hbm, acc_sh.at[pl.ds(s * RPT, RPT)])
        plsc.subcore_barrier()
        for cc in range(4):
            j = cc % NB
            wait_gather(j)
            scatter(j)
            jj = (j + 2) % NB
            if cc >= 2:
                wait_scatter(jj)
            load_and_gather(jj, cc + 2)

        def body(g, carry):
            base = 4 * g
            for j in range(NB):
                wait_gather(j)
                scatter(j)
                jj = (j + 2) % NB
                wait_scatter(jj)
                load_and_gather(jj, base + j + 2)
            return carry

        lax.fori_loop(1, (NCH - 5) // 4, body, 0)

        for t in range(5):
            cc = NCH - 5 + t
            j = cc % NB
            wait_gather(j)
            scatter(j)
            if t < 3:
                jj = (j + 2) % NB
                wait_scatter(jj)
                load_and_gather(jj, cc + 2)
        for j in (1, 2, 3, 0):
            wait_scatter(j)

        plsc.subcore_barrier()
        pltpu.sync_copy(acc_sh.at[pl.ds(s * RPT, RPT)],
                        out_hbm.at[c, pl.ds(s * RPT, RPT)])

    return k(eidx4, y, zeros_rows)


_RB = 1000
_NG = N // _RB


def _dinv_block(degp_b):
    return lax.rsqrt(degp_b[0] + degp_b[1] + 1.0)


def _tc_scale_matmul(x, W, degp3):
    def body(x_b, w_b, degp_b, y_b):
        dinv = _dinv_block(degp_b)
        y_b[...] = jnp.dot(x_b[...], w_b[...],
                           preferred_element_type=jnp.float32) * dinv

    return pl.pallas_call(
        body,
        grid=(_NG,),
        in_specs=[
            pl.BlockSpec((_RB, D), lambda i: (i, 0)),
            pl.BlockSpec((D, H), lambda i: (0, 0)),
            pl.BlockSpec((2, _RB, 1), lambda i: (0, i, 0)),
        ],
        out_specs=pl.BlockSpec((_RB, H), lambda i: (i, 0)),
        out_shape=jax.ShapeDtypeStruct((N, H), jnp.float32),
    )(x, W, degp3)


def _tc_conv1_out(acc, y1, degp3, b1, W2):
    def body(acc_b, y1_b, degp_b, b1_b, w2_b, y2_b):
        dinv = _dinv_block(degp_b)
        h = jnp.maximum(dinv * (acc_b[0] + acc_b[1] + y1_b[...]) + b1_b[...],
                        0.0)
        y2_b[...] = jnp.dot(h, w2_b[...],
                            preferred_element_type=jnp.float32) * dinv

    return pl.pallas_call(
        body,
        grid=(_NG,),
        in_specs=[
            pl.BlockSpec((2, _RB, H), lambda i: (0, i, 0)),
            pl.BlockSpec((_RB, H), lambda i: (i, 0)),
            pl.BlockSpec((2, _RB, 1), lambda i: (0, i, 0)),
            pl.BlockSpec((1, H), lambda i: (0, 0)),
            pl.BlockSpec((H, H), lambda i: (0, 0)),
        ],
        out_specs=pl.BlockSpec((_RB, H), lambda i: (i, 0)),
        out_shape=jax.ShapeDtypeStruct((N, H), jnp.float32),
    )(acc, y1, degp3, b1, W2)


def _tc_conv2_pool(acc, y2, degp3, b2, Wg, bg, batch2d, Wm, bm):
    def body(acc_r, y2_r, degp_r, b2_r, wg_r, bg_r, batch_r, wm_r, bm_r,
             mu_r):
        dinv = lax.rsqrt(degp_r[0, 0:N, :] + degp_r[1, 0:N, :] + 1.0)
        h2 = (dinv * (acc_r[0, 0:N, :] + acc_r[1, 0:N, :] + y2_r[...])
              + b2_r[...])
        g = jnp.dot(h2, wg_r[...],
                    preferred_element_type=jnp.float32) + bg_r[...]
        mask = (batch_r[...] ==
                lax.broadcasted_iota(jnp.int32, (N, B), 1))
        maskf = mask.astype(jnp.float32)
        gm = jnp.max(jnp.where(mask, g, -1e30), axis=0, keepdims=True)
        gmax_node = jnp.sum(maskf * gm, axis=1, keepdims=True)
        ge = jnp.exp(g - gmax_node)
        gs = jnp.sum(maskf * ge, axis=0, keepdims=True)
        gs_node = jnp.sum(maskf * gs, axis=1, keepdims=True)
        gate = ge / gs_node
        wgt = maskf * gate
        pooled = lax.dot_general(wgt, h2,
                                 (((0,), (0,)), ((), ())),
                                 preferred_element_type=jnp.float32)
        mu_r[...] = jnp.dot(pooled, wm_r[...],
                            preferred_element_type=jnp.float32) + bm_r[...]

    return pl.pallas_call(
        body,
        in_specs=[
            pl.BlockSpec((2, NP, H), lambda: (0, 0, 0)),
            pl.BlockSpec((N, H), lambda: (0, 0)),
            pl.BlockSpec((2, NP, 1), lambda: (0, 0, 0)),
            pl.BlockSpec((1, H), lambda: (0, 0)),
            pl.BlockSpec((H, 1), lambda: (0, 0)),
            pl.BlockSpec((1, 1), lambda: (0, 0)),
            pl.BlockSpec((N, 1), lambda: (0, 0)),
            pl.BlockSpec((H, L), lambda: (0, 0)),
            pl.BlockSpec((1, L), lambda: (0, 0)),
        ],
        out_specs=pl.BlockSpec((B, L), lambda: (0, 0)),
        out_shape=jax.ShapeDtypeStruct((B, L), jnp.float32),
    )(acc, y2, degp3, b2, Wg, bg, batch2d, Wm, bm)


def kernel(x, edge_index, batch, W1, b1, W2, b2, Wg, bg, Wm, bm, Wv, bv):
    del Wv, bv
    src3 = edge_index[0].reshape(NTILES, NCH, CH)
    dst3 = edge_index[1].reshape(NTILES, NCH, CH)
    eidx4 = jnp.stack([src3, dst3], axis=2)

    ones_c = jnp.ones((CH,), jnp.float32)
    zeros_r = jnp.zeros((RPT,), jnp.float32)
    zeros_rows = jnp.zeros((RPT, H), jnp.float32)

    degp = _sc_degree(dst3, ones_c, zeros_r)
    degp3 = degp.reshape(2, NP, 1)

    y1 = _tc_scale_matmul(x, W1, degp3)
    acc1 = _sc_edge_accum(eidx4, y1, zeros_rows)
    y2 = _tc_conv1_out(acc1, y1, degp3,
                       b1.reshape(1, H), W2)
    acc2 = _sc_edge_accum(eidx4, y2, zeros_rows)
    mu = _tc_conv2_pool(acc2, y2, degp3, b2.reshape(1, H),
                        Wg, bg.reshape(1, 1),
                        batch.reshape(N, 1), Wm, bm.reshape(1, L))
    return mu
